# baseline (speedup 1.0000x reference)
import functools

import jax
import jax.numpy as jnp
from jax import lax
from jax.experimental import pallas as pl
from jax.experimental.pallas import tpu as pltpu
from jax.experimental.pallas import tpu_sc as plsc

NUM_FACTORS = 32
MU = 3.5
BATCH = 16384
NC = 2
NS = 16
L = 16
NW = NC * NS
BPW = BATCH // NW
CHUNK = 128
NCHUNK = BPW // CHUNK
GRAN = 128
ROWS_PER_GRAN = GRAN // NUM_FACTORS


def _sc_body(uidx_hbm, iidx_hbm, uw_hbm, iw_hbm, ub_hbm, ib_hbm, out_hbm,
             uidx_v, iidx_v, utidx_v, itidx_v, ubuf_v, ibuf_v,
             ub_v, ib_v, out_v, sem):
    c = lax.axis_index("c")
    s = lax.axis_index("s")
    wid = s * NC + c

    pltpu.sync_copy(uidx_hbm.at[wid], uidx_v)
    pltpu.sync_copy(iidx_hbm.at[wid], iidx_v)

    bias_copies = []
    for j in range(NCHUNK):
        dst = pl.ds(j * CHUNK, CHUNK)
        bias_copies.append(
            pltpu.async_copy(ub_hbm.at[uidx_v.at[j]], ub_v.at[dst], sem))
        bias_copies.append(
            pltpu.async_copy(ib_hbm.at[iidx_v.at[j]], ib_v.at[dst], sem))

    lane = lax.iota(jnp.int32, L)

    def fill_tidx(j):
        def body(g, carry):
            sl = pl.ds(g * L, L)
            utidx_v[sl] = lax.shift_right_logical(uidx_v[j, sl], 2)
            itidx_v[sl] = lax.shift_right_logical(iidx_v[j, sl], 2)
            return carry
        lax.fori_loop(0, CHUNK // L, body, 0)

    def fire(j, buf_slot):
        return [
            pltpu.async_copy(uw_hbm.at[utidx_v], ubuf_v.at[buf_slot], sem),
            pltpu.async_copy(iw_hbm.at[itidx_v], ibuf_v.at[buf_slot], sem),
        ]

    fill_tidx(0)
    inflight = fire(0, 0)

    for j in range(NCHUNK):
        for cp in inflight:
            cp.wait()
        if j + 1 < NCHUNK:
            fill_tidx(j + 1)
            inflight = fire(j + 1, (j + 1) % 2)
        buf_slot = j % 2

        def group(g, carry):
            uiv = uidx_v[j, pl.ds(g * L, L)]
            iiv = iidx_v[j, pl.ds(g * L, L)]
            offu = (uiv & (ROWS_PER_GRAN - 1)) * NUM_FACTORS
            offi = (iiv & (ROWS_PER_GRAN - 1)) * NUM_FACTORS
            dots = jnp.zeros((L,), jnp.float32)
            for k in range(L):
                r = g * L + k
                ou = offu[k]
                oi = offi[k]
                u0 = ubuf_v[buf_slot, r, pl.ds(ou, L)]
                u1 = ubuf_v[buf_slot, r, pl.ds(ou + L, L)]
                i0 = ibuf_v[buf_slot, r, pl.ds(oi, L)]
                i1 = ibuf_v[buf_slot, r, pl.ds(oi + L, L)]
                v = u0 * i0 + u1 * i1
                h = v + lax.rev(v, (0,))
                sdot = (((h[0] + h[1]) + (h[2] + h[3]))
                        + ((h[4] + h[5]) + (h[6] + h[7])))
                dots = jnp.where(lane == k, sdot, dots)
            out_v[pl.ds(j * CHUNK + g * L, L)] = dots
            return carry

        lax.fori_loop(0, CHUNK // L, group, 0)

    for cp in bias_copies:
        cp.wait()

    def addbias(g, carry):
        sl = pl.ds(g * L, L)
        out_v[sl] = out_v[sl] + ub_v[sl] + ib_v[sl] + MU
        return carry

    lax.fori_loop(0, BPW // L, addbias, 0)

    pltpu.sync_copy(out_v, out_hbm.at[pl.ds(wid * BPW, BPW)])


@jax.jit
def _run(uidx3, iidx3, uw, iw, ubf, ibf):
    mesh = plsc.VectorSubcoreMesh(core_axis_name="c", subcore_axis_name="s")
    f = pl.kernel(
        _sc_body,
        mesh=mesh,
        compiler_params=pltpu.CompilerParams(use_tc_tiling_on_sc=False),
        out_type=jax.ShapeDtypeStruct((BATCH,), jnp.float32),
        scratch_types=[
            pltpu.VMEM((NCHUNK, CHUNK), jnp.int32),
            pltpu.VMEM((NCHUNK, CHUNK), jnp.int32),
            pltpu.VMEM((CHUNK,), jnp.int32),
            pltpu.VMEM((CHUNK,), jnp.int32),
            pltpu.VMEM((2, CHUNK, GRAN), jnp.float32),
            pltpu.VMEM((2, CHUNK, GRAN), jnp.float32),
            pltpu.VMEM((BPW,), jnp.float32),
            pltpu.VMEM((BPW,), jnp.float32),
            pltpu.VMEM((BPW,), jnp.float32),
            pltpu.SemaphoreType.DMA,
        ],
    )
    return f(uidx3, iidx3, uw, iw, ubf, ibf)


def kernel(user_idx, item_idx, embed_user_w, embed_item_w, user_bias_w, item_bias_w):
    uidx3 = user_idx.reshape(NW, NCHUNK, CHUNK)
    iidx3 = item_idx.reshape(NW, NCHUNK, CHUNK)
    uw = embed_user_w.reshape(-1, GRAN)
    iw = embed_item_w.reshape(-1, GRAN)
    ubf = user_bias_w.reshape(-1)
    ibf = item_bias_w.reshape(-1)
    return _run(uidx3, iidx3, uw, iw, ubf, ibf)


# speedup vs baseline: 1.0074x; 1.0074x over previous
import functools

import jax
import jax.numpy as jnp
from jax import lax
from jax.experimental import pallas as pl
from jax.experimental.pallas import tpu as pltpu
from jax.experimental.pallas import tpu_sc as plsc

NUM_FACTORS = 32
MU = 3.5
BATCH = 16384
NC = 2
NS = 16
L = 16
NW = NC * NS
BPW = BATCH // NW
CHUNK = 128
NCHUNK = BPW // CHUNK


def _sc_body(uidx_hbm, iidx_hbm, uw_hbm, iw_hbm, ub_hbm, ib_hbm, out_hbm,
             uidx_v, iidx_v, urows_v, irows_v, ub_v, ib_v, out_v, sem):
    c = lax.axis_index("c")
    s = lax.axis_index("s")
    wid = s * NC + c

    pltpu.sync_copy(uidx_hbm.at[wid], uidx_v)
    pltpu.sync_copy(iidx_hbm.at[wid], iidx_v)

    copies = []
    for j in range(NCHUNK):
        dst = pl.ds(j * CHUNK, CHUNK)
        copies.append(pltpu.async_copy(uw_hbm.at[uidx_v.at[j]], urows_v.at[dst], sem))
        copies.append(pltpu.async_copy(iw_hbm.at[iidx_v.at[j]], irows_v.at[dst], sem))
        copies.append(pltpu.async_copy(ub_hbm.at[uidx_v.at[j]], ub_v.at[dst], sem))
        copies.append(pltpu.async_copy(ib_hbm.at[iidx_v.at[j]], ib_v.at[dst], sem))
    for cp in copies:
        cp.wait()

    lane = lax.iota(jnp.int32, L)

    def group(g, carry):
        dots = jnp.zeros((L,), jnp.float32)
        for k in range(L):
            r = g * L + k
            u0 = urows_v[r, pl.ds(0, L)]
            u1 = urows_v[r, pl.ds(L, L)]
            i0 = irows_v[r, pl.ds(0, L)]
            i1 = irows_v[r, pl.ds(L, L)]
            v = u0 * i0 + u1 * i1
            h = v + lax.rev(v, (0,))
            s = (((h[0] + h[1]) + (h[2] + h[3]))
                 + ((h[4] + h[5]) + (h[6] + h[7])))
            dots = jnp.where(lane == k, s, dots)
        sl = pl.ds(g * L, L)
        out_v[sl] = dots + ub_v[sl] + ib_v[sl] + MU
        return carry

    lax.fori_loop(0, BPW // L, group, 0)

    pltpu.sync_copy(out_v, out_hbm.at[pl.ds(wid * BPW, BPW)])


@jax.jit
def _run(uidx3, iidx3, uw, iw, ubf, ibf):
    mesh = plsc.VectorSubcoreMesh(core_axis_name="c", subcore_axis_name="s")
    f = pl.kernel(
        _sc_body,
        mesh=mesh,
        compiler_params=pltpu.CompilerParams(use_tc_tiling_on_sc=False),
        out_type=jax.ShapeDtypeStruct((BATCH,), jnp.float32),
        scratch_types=[
            pltpu.VMEM((NCHUNK, CHUNK), jnp.int32),
            pltpu.VMEM((NCHUNK, CHUNK), jnp.int32),
            pltpu.VMEM((BPW, NUM_FACTORS), jnp.float32),
            pltpu.VMEM((BPW, NUM_FACTORS), jnp.float32),
            pltpu.VMEM((BPW,), jnp.float32),
            pltpu.VMEM((BPW,), jnp.float32),
            pltpu.VMEM((BPW,), jnp.float32),
            pltpu.SemaphoreType.DMA,
        ],
    )
    return f(uidx3, iidx3, uw, iw, ubf, ibf)


def kernel(user_idx, item_idx, embed_user_w, embed_item_w, user_bias_w, item_bias_w):
    uidx3 = user_idx.reshape(NW, NCHUNK, CHUNK)
    iidx3 = item_idx.reshape(NW, NCHUNK, CHUNK)
    ubf = user_bias_w.reshape(-1)
    ibf = item_bias_w.reshape(-1)
    return _run(uidx3, iidx3, embed_user_w, embed_item_w, ubf, ibf)


# speedup vs baseline: 1.0109x; 1.0034x over previous
import jax
import jax.numpy as jnp
from jax import lax
from jax.experimental import pallas as pl
from jax.experimental.pallas import tpu as pltpu
from jax.experimental.pallas import tpu_sc as plsc

NUM_FACTORS = 32
MU = 3.5
BATCH = 16384
NC = 2
NS = 16
L = 16
NW = NC * NS
BPW = BATCH // NW
CHUNK = 128
NCHUNK = BPW // CHUNK


def _sc_body(uidx_hbm, iidx_hbm, uw_hbm, iw_hbm, ub_hbm, ib_hbm, out_hbm,
             uidx_v, iidx_v, urows_v, irows_v, ub_v, ib_v, out_v, sem):
    c = lax.axis_index("c")
    s = lax.axis_index("s")
    wid = s * NC + c

    pltpu.sync_copy(uidx_hbm.at[wid], uidx_v)
    pltpu.sync_copy(iidx_hbm.at[wid], iidx_v)

    copies = []
    for j in range(NCHUNK):
        dst = pl.ds(j * CHUNK, CHUNK)
        copies.append(pltpu.async_copy(uw_hbm.at[uidx_v.at[j]], urows_v.at[dst], sem))
        copies.append(pltpu.async_copy(iw_hbm.at[iidx_v.at[j]], irows_v.at[dst], sem))
        copies.append(pltpu.async_copy(ub_hbm.at[uidx_v.at[j]], ub_v.at[dst], sem))
        copies.append(pltpu.async_copy(ib_hbm.at[iidx_v.at[j]], ib_v.at[dst], sem))
    for cp in copies:
        cp.wait()

    lane = lax.iota(jnp.int32, L)

    def group(g, carry):
        dots = jnp.zeros((L,), jnp.float32)
        for k in range(L):
            r = g * L + k
            u0 = urows_v[r, pl.ds(0, L)]
            u1 = urows_v[r, pl.ds(L, L)]
            i0 = irows_v[r, pl.ds(0, L)]
            i1 = irows_v[r, pl.ds(L, L)]
            v = u0 * i0 + u1 * i1
            h = v + lax.rev(v, (0,))
            s = (((h[0] + h[1]) + (h[2] + h[3]))
                 + ((h[4] + h[5]) + (h[6] + h[7])))
            dots = jnp.where(lane == k, s, dots)
        sl = pl.ds(g * L, L)
        out_v[sl] = dots + ub_v[sl] + ib_v[sl] + MU
        return carry

    lax.fori_loop(0, BPW // L, group, 0)

    pltpu.sync_copy(out_v, out_hbm.at[pl.ds(wid * BPW, BPW)])


@jax.jit
def _run(uidx3, iidx3, uw, iw, ubf, ibf):
    mesh = plsc.VectorSubcoreMesh(core_axis_name="c", subcore_axis_name="s")
    f = pl.kernel(
        _sc_body,
        mesh=mesh,
        compiler_params=pltpu.CompilerParams(use_tc_tiling_on_sc=False),
        out_type=jax.ShapeDtypeStruct((BATCH,), jnp.float32),
        scratch_types=[
            pltpu.VMEM((NCHUNK, CHUNK), jnp.int32),
            pltpu.VMEM((NCHUNK, CHUNK), jnp.int32),
            pltpu.VMEM((BPW, NUM_FACTORS), jnp.float32),
            pltpu.VMEM((BPW, NUM_FACTORS), jnp.float32),
            pltpu.VMEM((BPW,), jnp.float32),
            pltpu.VMEM((BPW,), jnp.float32),
            pltpu.VMEM((BPW,), jnp.float32),
            pltpu.SemaphoreType.DMA,
        ],
    )
    return f(uidx3, iidx3, uw, iw, ubf, ibf)


def kernel(user_idx, item_idx, embed_user_w, embed_item_w, user_bias_w, item_bias_w):
    uidx3 = user_idx.reshape(NW, NCHUNK, CHUNK)
    iidx3 = item_idx.reshape(NW, NCHUNK, CHUNK)
    ubf = user_bias_w.reshape(-1)
    ibf = item_bias_w.reshape(-1)
    return _run(uidx3, iidx3, embed_user_w, embed_item_w, ubf, ibf)
